# baseline (speedup 1.0000x reference)
import functools

import jax
import jax.numpy as jnp
from jax import lax
from jax.experimental import pallas as pl
from jax.experimental.pallas import tpu as pltpu
from jax.experimental.pallas import tpu_sc as plsc

B = 1024
NV = 2000
NVP = 2048
NC = 8000
K = 3

NW = 32
CPW = NC // NW
CHUNK = 16
NCHUNK = 16
ROWS = CHUNK * K


_TBLK = 512


def _table_body(w_ref, t_ref):
    x = jax.nn.sigmoid(w_ref[...])
    xt = x.T
    t_ref[...] = jnp.stack([1.0 - xt, xt]).astype(jnp.bfloat16)


def _build_table(w):
    nblk = NVP // _TBLK
    t3 = pl.pallas_call(
        _table_body,
        grid=(nblk,),
        in_specs=[pl.BlockSpec((B, _TBLK), lambda i: (0, i))],
        out_specs=pl.BlockSpec((2, _TBLK, B), lambda i: (0, i, 0)),
        out_shape=jax.ShapeDtypeStruct((2, NVP, B), jnp.bfloat16),
    )(w)
    return t3.reshape(2 * NVP, B)


_MESH = plsc.VectorSubcoreMesh(
    core_axis_name="c", subcore_axis_name="s", num_cores=2, num_subcores=16
)


@functools.partial(
    pl.kernel,
    out_type=jax.ShapeDtypeStruct((NW, 1, B), jnp.bfloat16),
    mesh=_MESH,
    scratch_types=[
        pltpu.VMEM((NCHUNK, ROWS), jnp.int32),
        pltpu.VMEM((2, ROWS, B // 2), jnp.int32),
        pltpu.VMEM((1, B), jnp.bfloat16),
        pltpu.SemaphoreType.DMA,
        pltpu.SemaphoreType.DMA,
    ],
    compiler_params=pltpu.CompilerParams(
        use_tc_tiling_on_sc=True, needs_layout_passes=False
    ),
)
def _clause_partials(tab_hbm, idx_hbm, out_hbm, idx_v, rows_v, acc_v, sem0, sem1):
    wid = lax.axis_index("s") * 2 + lax.axis_index("c")
    sems = (sem0, sem1)

    pltpu.sync_copy(idx_hbm.at[wid], idx_v)

    @pl.loop(0, B // 32)
    def _(g):
        acc_v[0, pl.ds(g * 32, 32)] = jnp.full((32,), 1.0, jnp.bfloat16)

    dmas = [None, None]
    dmas[0] = pltpu.async_copy(tab_hbm.at[idx_v.at[0]], rows_v.at[0], sems[0])
    for c in range(NCHUNK):
        cur = c & 1
        if c + 1 < NCHUNK:
            nxt = (c + 1) & 1
            dmas[nxt] = pltpu.async_copy(
                tab_hbm.at[idx_v.at[c + 1]], rows_v.at[nxt], sems[nxt]
            )
        dmas[cur].wait()
        n = CHUNK if c < NCHUNK - 1 else CPW - (NCHUNK - 1) * CHUNK

        @pl.loop(0, B // 32)
        def _(g):
            sl = pl.ds(g * 32, 32)
            p = [None] * 4
            one = jnp.full((32,), 1.0, jnp.bfloat16)
            sl16 = pl.ds(g * 16, 16)
            for i in range(n):
                r0 = plsc.bitcast(rows_v[cur, 3 * i, sl16], jnp.bfloat16)
                r1 = plsc.bitcast(rows_v[cur, 3 * i + 1, sl16], jnp.bfloat16)
                r2 = plsc.bitcast(rows_v[cur, 3 * i + 2, sl16], jnp.bfloat16)
                f = one - r0 * r1 * r2
                j = i & 3
                p[j] = f if p[j] is None else p[j] * f
            q01 = p[0] * p[1]
            q23 = p[2] if p[3] is None else p[2] * p[3]
            acc_v[0, sl] = acc_v[0, sl] * (q01 * q23)

    pltpu.sync_copy(acc_v, out_hbm.at[wid])


def _finish_body(p_ref, i_ref, o_ref):
    a = p_ref[...].astype(jnp.float32)
    while a.shape[0] > 1:
        h = a.shape[0] // 2
        a = a[:h] * a[h:]
    b_idx = i_ref[...]
    iot = lax.broadcasted_iota(jnp.int32, (B, B), 1)
    mat = jnp.where(b_idx == iot, a, 1.0)
    while mat.shape[1] > 1:
        h = mat.shape[1] // 2
        mat = mat[:, :h] * mat[:, h:]
    o_ref[...] = mat


def _finish(partials, inp2d):
    return pl.pallas_call(
        _finish_body,
        out_shape=jax.ShapeDtypeStruct((B, 1), jnp.float32),
    )(partials, inp2d)


def kernel(input, emb_weight, clause_idx, clause_sign):
    table = _build_table(emb_weight)

    flat = (clause_idx.ravel() + jnp.where(clause_sign.ravel() > 0.0, 0, NVP)).astype(jnp.int32)
    idx = flat.reshape(NW, CPW * K)
    idx = jnp.pad(idx, ((0, 0), (0, NCHUNK * ROWS - CPW * K)))
    idx = idx.reshape(NW, NCHUNK, ROWS)

    table_i32 = lax.bitcast_convert_type(table.reshape(2 * NVP, B // 2, 2), jnp.int32)
    partials = _clause_partials(table_i32, idx).reshape(NW, B)
    out = _finish(partials, input.reshape(B, 1).astype(jnp.int32))
    return out.reshape(B)


# speedup vs baseline: 1.6813x; 1.6813x over previous
import functools

import jax
import jax.numpy as jnp
from jax import lax
from jax.experimental import pallas as pl
from jax.experimental.pallas import tpu as pltpu
from jax.experimental.pallas import tpu_sc as plsc

B = 1024
NV = 2000
NVP = 2048
NC = 8000
K = 3

NW = 32
CPW = NC // NW
CHUNK = 16
NCHUNK = 16
ROWS = CHUNK * K


_TBLK = 512


def _table_body(w_ref, t_ref):
    x = jax.nn.sigmoid(w_ref[...])
    xt = x.T
    t_ref[...] = jnp.stack([1.0 - xt, xt])


def _build_table(w):
    nblk = NVP // _TBLK
    t3 = pl.pallas_call(
        _table_body,
        grid=(nblk,),
        in_specs=[pl.BlockSpec((B, _TBLK), lambda i: (0, i))],
        out_specs=pl.BlockSpec((2, _TBLK, B), lambda i: (0, i, 0)),
        out_shape=jax.ShapeDtypeStruct((2, NVP, B), jnp.float32),
    )(w)
    return t3.reshape(2 * NVP, B)


_MESH = plsc.VectorSubcoreMesh(
    core_axis_name="c", subcore_axis_name="s", num_cores=2, num_subcores=16
)


@functools.partial(
    pl.kernel,
    out_type=jax.ShapeDtypeStruct((NW, B), jnp.float32),
    mesh=_MESH,
    scratch_types=[
        pltpu.VMEM((NCHUNK, ROWS), jnp.int32),
        pltpu.VMEM((2, ROWS, B), jnp.float32),
        pltpu.VMEM((B,), jnp.float32),
        pltpu.SemaphoreType.DMA,
        pltpu.SemaphoreType.DMA,
    ],
    compiler_params=pltpu.CompilerParams(use_tc_tiling_on_sc=True),
)
def _clause_partials(tab_hbm, idx_hbm, out_hbm, idx_v, rows_v, acc_v, sem0, sem1):
    wid = lax.axis_index("s") * 2 + lax.axis_index("c")
    sems = (sem0, sem1)

    pltpu.sync_copy(idx_hbm.at[wid], idx_v)

    @pl.loop(0, B // 16)
    def _(g):
        acc_v[pl.ds(g * 16, 16)] = jnp.full((16,), 1.0, jnp.float32)

    dmas = [None, None]
    dmas[0] = pltpu.async_copy(tab_hbm.at[idx_v.at[0]], rows_v.at[0], sems[0])
    for c in range(NCHUNK):
        cur = c & 1
        if c + 1 < NCHUNK:
            nxt = (c + 1) & 1
            dmas[nxt] = pltpu.async_copy(
                tab_hbm.at[idx_v.at[c + 1]], rows_v.at[nxt], sems[nxt]
            )
        dmas[cur].wait()
        n = CHUNK if c < NCHUNK - 1 else CPW - (NCHUNK - 1) * CHUNK

        @plsc.parallel_loop(0, B // 16)
        def _(g):
            sl = pl.ds(g * 16, 16)
            p = [None] * 4
            for i in range(n):
                z = (
                    rows_v[cur, 3 * i, sl]
                    * rows_v[cur, 3 * i + 1, sl]
                    * rows_v[cur, 3 * i + 2, sl]
                )
                f = 1.0 - z
                j = i & 3
                p[j] = f if p[j] is None else p[j] * f
            q01 = p[0] * p[1]
            q23 = p[2] if p[3] is None else p[2] * p[3]
            acc_v[sl] = acc_v[sl] * (q01 * q23)

    pltpu.sync_copy(acc_v, out_hbm.at[wid])


def _finish_body(p_ref, i_ref, o_ref):
    a = p_ref[...]
    while a.shape[0] > 1:
        h = a.shape[0] // 2
        a = a[:h] * a[h:]
    b_idx = i_ref[...]
    iot = lax.broadcasted_iota(jnp.int32, (B, B), 1)
    mat = jnp.where(b_idx == iot, a, 1.0)
    while mat.shape[1] > 1:
        h = mat.shape[1] // 2
        mat = mat[:, :h] * mat[:, h:]
    o_ref[...] = mat


def _finish(partials, inp2d):
    return pl.pallas_call(
        _finish_body,
        out_shape=jax.ShapeDtypeStruct((B, 1), jnp.float32),
    )(partials, inp2d)


def kernel(input, emb_weight, clause_idx, clause_sign):
    table = _build_table(emb_weight)

    flat = (clause_idx.ravel() + jnp.where(clause_sign.ravel() > 0.0, 0, NVP)).astype(jnp.int32)
    idx = flat.reshape(NW, CPW * K)
    idx = jnp.pad(idx, ((0, 0), (0, NCHUNK * ROWS - CPW * K)))
    idx = idx.reshape(NW, NCHUNK, ROWS)

    partials = _clause_partials(table, idx)
    out = _finish(partials, input.reshape(B, 1).astype(jnp.int32))
    return out.reshape(B)
